# baseline (speedup 1.0000x reference)
import jax
import jax.numpy as jnp
import numpy as np
from jax.experimental import pallas as pl

_B, _N, _S = 1024, 16, 32
_SD, _ED, _HE = 512, 64, 512
_BM = 1024


def _coal_weight():
    pkey = jax.random.key(42)
    keys = jax.random.split(pkey, _B * _S)
    perms = jax.vmap(lambda k: jax.random.permutation(k, _N))(keys)
    inv = jnp.argsort(perms, axis=-1).reshape(_B, _S, _N)
    freq = jax.nn.one_hot(inv, _N, dtype=jnp.float32).sum(axis=1) / _S
    mask = (jnp.arange(_N) < _N - 1).astype(jnp.float32)[None, :, None]
    return (freq * mask).reshape(_B, _N * _N)


_WC = np.asarray(jax.jit(_coal_weight)())


def _hpq_kernel(st_ref, rqt_ref, wc_ref, w1a_ref, b1a_ref, w1b_ref, b1b_ref,
                wb1t_ref, bb1_ref, wfa_ref, bfa_ref, wfbt_ref, bfb_ref,
                wb2at_ref, bb2a_ref, wb2bt_ref, bb2b_ref, out_ref):
    f32 = jnp.float32
    st = st_ref[...]
    wb1 = wb1t_ref[...].T
    wfb = wfbt_ref[...].T
    wb2a = wb2at_ref[...].T
    h1 = jax.nn.gelu(jnp.dot(st, w1a_ref[...], preferred_element_type=f32) + b1a_ref[...])
    w1 = jnp.dot(h1, w1b_ref[...], preferred_element_type=f32) + b1b_ref[...]
    b1 = jnp.dot(st, wb1, preferred_element_type=f32) + bb1_ref[...]
    hf = jax.nn.gelu(jnp.dot(st, wfa_ref[...], preferred_element_type=f32) + bfa_ref[...])
    wf = jnp.dot(hf, wfb, preferred_element_type=f32) + bfb_ref[...]
    hb = jax.nn.gelu(jnp.dot(st, wb2a, preferred_element_type=f32) + bb2a_ref[...])
    b2 = jnp.sum(hb * wb2bt_ref[...], axis=1, keepdims=True) + bb2b_ref[...]

    rq = rqt_ref[...].T
    wc = wc_ref[...]

    r16 = jax.lax.broadcasted_iota(jnp.int32, (_N * _N, _N), 0) // _N
    c16 = jax.lax.broadcasted_iota(jnp.int32, (_N * _N, _N), 1)
    g16 = (r16 == c16).astype(f32)
    rq_t = jnp.tile(rq, (1, _N))
    coal = jnp.dot(wc * rq_t, g16, preferred_element_type=f32)

    w1_0 = w1[:, :_ED]
    w1_1 = w1[:, _ED:]
    cols = []
    for n in range(_N):
        pre = coal[:, n:n + 1] * w1_0 + rq[:, n:n + 1] * w1_1 + b1
        cols.append(jnp.sum(jax.nn.gelu(pre) * wf, axis=1, keepdims=True))
    y = jnp.concatenate(cols, axis=1) + b2
    out_ref[...] = jnp.abs(y).T


def kernel(states, agent_qs, w1a, b1a, w1b, b1b, wb1, bb1, wfa, bfa, wfb, bfb,
           wb2a, bb2a, wb2b, bb2b):
    rqt = agent_qs[:, :, 0].T
    wc = jnp.asarray(_WC)
    row = lambda i: (i, 0)
    col = lambda i: (0, i)
    rep = lambda i: (0, 0)
    outt = pl.pallas_call(
        _hpq_kernel,
        grid=(_B // _BM,),
        in_specs=[
            pl.BlockSpec((_BM, _SD), row),
            pl.BlockSpec((_N, _BM), col),
            pl.BlockSpec((_BM, _N * _N), row),
            pl.BlockSpec((_SD, _HE), rep),
            pl.BlockSpec((1, _HE), rep),
            pl.BlockSpec((_HE, 2 * _ED), rep),
            pl.BlockSpec((1, 2 * _ED), rep),
            pl.BlockSpec((_ED, _SD), rep),
            pl.BlockSpec((1, _ED), rep),
            pl.BlockSpec((_SD, _HE), rep),
            pl.BlockSpec((1, _HE), rep),
            pl.BlockSpec((_ED, _SD), rep),
            pl.BlockSpec((1, _ED), rep),
            pl.BlockSpec((_ED, _SD), rep),
            pl.BlockSpec((1, _ED), rep),
            pl.BlockSpec((1, _ED), rep),
            pl.BlockSpec((1, 1), rep),
        ],
        out_specs=pl.BlockSpec((_N, _BM), col),
        out_shape=jax.ShapeDtypeStruct((_N, _B), jnp.float32),
    )(states, rqt, wc, w1a, b1a.reshape(1, _HE), w1b, b1b.reshape(1, 2 * _ED),
      wb1.T, bb1.reshape(1, _ED), wfa, bfa.reshape(1, _HE), wfb.T,
      bfb.reshape(1, _ED), wb2a.T, bb2a.reshape(1, _ED), wb2b.reshape(1, _ED),
      bb2b.reshape(1, 1))
    return outt.T


# speedup vs baseline: 1.7362x; 1.7362x over previous
import jax
import jax.numpy as jnp
import numpy as np
from jax.experimental import pallas as pl

_B, _N, _S = 1024, 16, 32
_SD, _ED, _HE = 512, 64, 512
_BM = 1024


def _coal_weight():
    pkey = jax.random.key(42)
    keys = jax.random.split(pkey, _B * _S)
    perms = jax.vmap(lambda k: jax.random.permutation(k, _N))(keys)
    inv = jnp.argsort(perms, axis=-1).reshape(_B, _S, _N)
    freq = jax.nn.one_hot(inv, _N, dtype=jnp.float32).sum(axis=1) / _S
    mask = (jnp.arange(_N) < _N - 1).astype(jnp.float32)[None, :, None]
    return (freq * mask).reshape(_B, _N * _N)


_WC = np.asarray(jax.jit(_coal_weight)())


def _hpq_kernel(st_ref, rqt_ref, wc_ref, w1a_ref, b1a_ref, w1b_ref, b1b_ref,
                wb1t_ref, bb1_ref, wfa_ref, bfa_ref, wfbt_ref, bfb_ref,
                wb2at_ref, bb2a_ref, wb2bt_ref, bb2b_ref, out_ref):
    f32 = jnp.float32
    st = st_ref[...]
    wb1 = wb1t_ref[...].T
    wfb = wfbt_ref[...].T
    wb2a = wb2at_ref[...].T
    h1 = jax.nn.gelu(jnp.dot(st, w1a_ref[...], preferred_element_type=f32) + b1a_ref[...])
    w1 = jnp.dot(h1, w1b_ref[...], preferred_element_type=f32) + b1b_ref[...]
    b1 = jnp.dot(st, wb1, preferred_element_type=f32) + bb1_ref[...]
    hf = jax.nn.gelu(jnp.dot(st, wfa_ref[...], preferred_element_type=f32) + bfa_ref[...])
    wf = jnp.dot(hf, wfb, preferred_element_type=f32) + bfb_ref[...]
    hb = jax.nn.gelu(jnp.dot(st, wb2a, preferred_element_type=f32) + bb2a_ref[...])
    b2 = jnp.sum(hb * wb2bt_ref[...], axis=1, keepdims=True) + bb2b_ref[...]

    rq = rqt_ref[...].T
    wc = wc_ref[...]

    r16 = jax.lax.broadcasted_iota(jnp.int32, (_N * _N, _N), 0) // _N
    c16 = jax.lax.broadcasted_iota(jnp.int32, (_N * _N, _N), 1)
    g16 = (r16 == c16).astype(f32)
    rr = jax.lax.broadcasted_iota(jnp.int32, (_N, _N * _ED), 1) // _ED
    cr = jax.lax.broadcasted_iota(jnp.int32, (_N, _N * _ED), 0)
    e64 = (rr == cr).astype(f32)
    r64 = jax.lax.broadcasted_iota(jnp.int32, (_N * _ED, _N), 0) // _ED
    c64 = jax.lax.broadcasted_iota(jnp.int32, (_N * _ED, _N), 1)
    g64 = (r64 == c64).astype(f32)

    rq_t = jnp.tile(rq, (1, _N))
    coal = jnp.dot(wc * rq_t, g16, preferred_element_type=f32)

    w1_0 = jnp.tile(w1[:, :_ED], (1, _N))
    w1_1 = jnp.tile(w1[:, _ED:], (1, _N))
    b1_t = jnp.tile(b1, (1, _N))
    wf_t = jnp.tile(wf, (1, _N))
    coal_r = jnp.dot(coal, e64, preferred_element_type=f32)
    rq_r = jnp.dot(rq, e64, preferred_element_type=f32)
    hidden = jax.nn.gelu(coal_r * w1_0 + rq_r * w1_1 + b1_t)
    y = jnp.dot(hidden * wf_t, g64, preferred_element_type=f32) + b2
    out_ref[...] = jnp.abs(y).T


def kernel(states, agent_qs, w1a, b1a, w1b, b1b, wb1, bb1, wfa, bfa, wfb, bfb,
           wb2a, bb2a, wb2b, bb2b):
    rqt = agent_qs[:, :, 0].T
    wc = jnp.asarray(_WC)
    row = lambda i: (i, 0)
    col = lambda i: (0, i)
    rep = lambda i: (0, 0)
    outt = pl.pallas_call(
        _hpq_kernel,
        grid=(_B // _BM,),
        in_specs=[
            pl.BlockSpec((_BM, _SD), row),
            pl.BlockSpec((_N, _BM), col),
            pl.BlockSpec((_BM, _N * _N), row),
            pl.BlockSpec((_SD, _HE), rep),
            pl.BlockSpec((1, _HE), rep),
            pl.BlockSpec((_HE, 2 * _ED), rep),
            pl.BlockSpec((1, 2 * _ED), rep),
            pl.BlockSpec((_ED, _SD), rep),
            pl.BlockSpec((1, _ED), rep),
            pl.BlockSpec((_SD, _HE), rep),
            pl.BlockSpec((1, _HE), rep),
            pl.BlockSpec((_ED, _SD), rep),
            pl.BlockSpec((1, _ED), rep),
            pl.BlockSpec((_ED, _SD), rep),
            pl.BlockSpec((1, _ED), rep),
            pl.BlockSpec((1, _ED), rep),
            pl.BlockSpec((1, 1), rep),
        ],
        out_specs=pl.BlockSpec((_N, _BM), col),
        out_shape=jax.ShapeDtypeStruct((_N, _B), jnp.float32),
    )(states, rqt, wc, w1a, b1a.reshape(1, _HE), w1b, b1b.reshape(1, 2 * _ED),
      wb1.T, bb1.reshape(1, _ED), wfa, bfa.reshape(1, _HE), wfb.T,
      bfb.reshape(1, _ED), wb2a.T, bb2a.reshape(1, _ED), wb2b.reshape(1, _ED),
      bb2b.reshape(1, 1))
    return outt.T
